# Initial kernel scaffold; baseline (speedup 1.0000x reference)
import jax
import jax.numpy as jnp
from jax.experimental import pallas as pl


def kernel(face_normals, t_pos_idx):
    raise NotImplementedError("write your pallas kernel here")


# speedup vs baseline: 1.2934x; 1.2934x over previous
import jax
import jax.numpy as jnp
from jax.experimental import pallas as pl
from jax.experimental.pallas import tpu as pltpu

_F = 200000
_E = 3 * _F
_EPAD = 614400


def _finish_kernel(x0, y0, z0, x1, y1, z1, m, out_ref):
    dot = x0[...] * x1[...] + y0[...] * y1[...] + z0[...] * z1[...]
    term = 1.0 - jnp.clip(dot, -1.0, 1.0)
    out_ref[0, 0] = jnp.sum(jnp.abs(term) * m[...])


def kernel(face_normals, t_pos_idx):
    a = t_pos_idx.astype(jnp.int64)
    v0, v1, v2 = a[:, 0], a[:, 1], a[:, 2]
    ea = jnp.concatenate([v0, v1, v2])
    eb = jnp.concatenate([v1, v2, v0])
    c0 = jnp.minimum(ea, eb)
    c1 = jnp.maximum(ea, eb)
    order = (ea > eb).astype(jnp.int64)
    f = jnp.arange(_F, dtype=jnp.int64)
    e = jnp.concatenate([3 * f, 3 * f + 1, 3 * f + 2])
    key = c0 * 131072 + c1
    packed = key * (1 << 21) + order * (1 << 20) + e
    s = jnp.sort(packed)
    key_s = s >> 21
    order_s = (s >> 20) & 1
    e_s = s & ((1 << 20) - 1)
    head = jnp.concatenate([jnp.ones((1,), jnp.bool_), key_s[1:] != key_s[:-1]])
    num_u = jnp.sum(head)
    gid = jnp.cumsum(head) - 1
    t0e = jax.ops.segment_max(jnp.where(order_s == 0, e_s, -1), gid, num_segments=_E)
    t1e = jax.ops.segment_max(jnp.where(order_s == 1, e_s, -1), gid, num_segments=_E)
    t0 = jnp.where(t0e >= 0, t0e // 3, 0)
    t1 = jnp.where(t1e >= 0, t1e // 3, 0)
    n0 = face_normals[t0]
    n1 = face_normals[t1]
    mask = (jnp.arange(_E) < num_u).astype(jnp.float32)

    def pad2d(x):
        return jnp.pad(x, (0, _EPAD - _E)).reshape(_EPAD // 128, 128)

    args = [pad2d(n0[:, 0]), pad2d(n0[:, 1]), pad2d(n0[:, 2]),
            pad2d(n1[:, 0]), pad2d(n1[:, 1]), pad2d(n1[:, 2]), pad2d(mask)]
    total = pl.pallas_call(
        _finish_kernel,
        out_shape=jax.ShapeDtypeStruct((1, 1), jnp.float32),
        out_specs=pl.BlockSpec(memory_space=pltpu.SMEM),
    )(*args)
    return (total[0, 0] / num_u).astype(jnp.float32)


# speedup vs baseline: 43.0700x; 33.2999x over previous
import numpy as np
import jax
import jax.numpy as jnp
from jax import lax
from jax.experimental import pallas as pl
from jax.experimental.pallas import tpu as pltpu
from jax.experimental.pallas import tpu_sc as plsc

_F = 200000
_E = 3 * _F
_NW = 16
_PER_W = _E // _NW
_W = 128
_CAP = 37888
_M = 1 << 20
_TAGSH = 1 << 20
_R = 6

_i32 = jnp.int32


def _c(x):
    return int(np.int32(np.uint32(x)))


_KA = _c(0x9E3779B1)
_KB = _c(0x85EBCA6B)
_KC = _c(0x27D4EB2F)
_KD = _c(0x7FEB352D)
_KE = _c(0x846CA68B)


def _hash(c0, c1, r):
    x = c0 * _KA + c1 * _KB + r * _KC
    x = x ^ lax.shift_right_logical(x, _i32(16))
    x = x * _KD
    x = x ^ lax.shift_right_logical(x, _i32(13))
    x = x * _KE
    x = x ^ lax.shift_right_logical(x, _i32(16))
    return jnp.bitwise_and(x, _M - 1)


def _body(c0f, c1f, c0p, c1p, tvp, fnx, fny, fnz, out,
          T, T0, T1, ag, ac0, ac1, atv, slots, wslots,
          b_c0, b_c1, b_g, b_tv, b_slot, b_val, b_w, b_w2,
          b_i1, b_i2, b_i3, b_i4, b_i5, b_i6,
          b_x0, b_y0, b_z0, b_x1, b_y1, b_z1,
          b_red, b_red2, zbuf, shared,
          s0, s1, s2, s3, s4, s5):
    wid = lax.axis_index("s").astype(_i32)
    LANE = jnp.arange(16, dtype=_i32)
    NV = _W // 16

    def zb(j, _):
        zbuf[pl.ds(j * 16, 16)] = jnp.zeros((16,), _i32)
        return 0
    lax.fori_loop(_i32(0), _i32(8192 // 16), zb, 0)
    stripe = _M // _NW
    nz = stripe // 8192

    def zt(j, _):
        base = wid * stripe + j * 8192
        pltpu.sync_copy(zbuf, T.at[pl.ds(base, 8192)])
        pltpu.sync_copy(zbuf, T0.at[pl.ds(base, 8192)])
        pltpu.sync_copy(zbuf, T1.at[pl.ds(base, 8192)])
        return 0
    lax.fori_loop(_i32(0), _i32(nz), zt, 0)
    plsc.subcore_barrier()

    def phase_a(src_c0, src_c1, src_g, base_off, n, r, tag):
        nwin = (n + _W - 1) // _W

        def win(w, _):
            off = w * _W
            pltpu.sync_copy(src_c0.at[pl.ds(base_off + off, _W)], b_c0)
            pltpu.sync_copy(src_c1.at[pl.ds(base_off + off, _W)], b_c1)
            if src_g is not None:
                pltpu.sync_copy(src_g.at[pl.ds(base_off + off, _W)], b_g)
            rem = n - off

            def vec(j, _):
                sl = pl.ds(j * 16, 16)
                c0v = b_c0[sl]
                c1v = b_c1[sl]
                lane = j * 16 + LANE
                mask = lane < rem
                sv = _hash(c0v, c1v, r)
                if src_g is None:
                    gidv = wid * _PER_W + off + lane
                else:
                    gidv = b_g[sl]
                b_slot[sl] = jnp.where(mask, sv, -1)
                b_val[sl] = tag + gidv + 1
                return 0
            lax.fori_loop(_i32(0), _i32(NV), vec, 0)
            pltpu.sync_copy(b_slot, slots.at[pl.ds(wid * _CAP + off, _W)])
            pltpu.async_copy(
                b_val, T.at[plsc.Indices(b_slot, ignored_value=-1)], s0
            ).wait()
            return 0
        lax.fori_loop(_i32(0), nwin, win, 0)

    def phase_bc(src_c0, src_c1, src_tv, src_g, base_off, n, r, tag):
        nwin = (n + _W - 1) // _W

        def win(w, carry):
            newn, wn = carry
            off = w * _W
            pltpu.sync_copy(src_c0.at[pl.ds(base_off + off, _W)], b_c0)
            pltpu.sync_copy(src_c1.at[pl.ds(base_off + off, _W)], b_c1)
            pltpu.sync_copy(src_tv.at[pl.ds(base_off + off, _W)], b_tv)
            if src_g is not None:
                pltpu.sync_copy(src_g.at[pl.ds(base_off + off, _W)], b_g)
            pltpu.sync_copy(slots.at[pl.ds(wid * _CAP + off, _W)], b_slot)
            pltpu.async_copy(
                T.at[plsc.Indices(b_slot, ignored_value=-1)], b_w, s0
            ).wait()
            rem = n - off

            def v1(j, _):
                sl = pl.ds(j * 16, 16)
                wv = b_w[sl]
                widv = wv - (tag + 1)
                lane = j * 16 + LANE
                ok = (lane < rem) & (widv >= 0) & (widv < _E)
                b_i1[sl] = jnp.where(ok, widv, -1)
                if src_g is None:
                    b_g[sl] = wid * _PER_W + off + lane
                return 0
            lax.fori_loop(_i32(0), _i32(NV), v1, 0)
            d1 = pltpu.async_copy(
                c0f.at[plsc.Indices(b_i1, ignored_value=-1)], b_w2, s1)
            d2 = pltpu.async_copy(
                c1f.at[plsc.Indices(b_i1, ignored_value=-1)], b_i6, s2)
            d1.wait()
            d2.wait()

            def v2(j, carry2):
                nn, wc0 = carry2
                sl = pl.ds(j * 16, 16)
                c0v = b_c0[sl]
                c1v = b_c1[sl]
                tvv = b_tv[sl]
                gidv = b_g[sl]
                slotv = b_slot[sl]
                widv = b_i1[sl]
                c0wv = b_w2[sl]
                c1wv = b_i6[sl]
                lane = j * 16 + LANE
                mask = lane < rem
                res = (widv >= 0) & (c0wv == c0v) & (c1wv == c1v) & mask
                iswin = res & (widv == gidv)
                ordv = jnp.bitwise_and(tvv, 1)
                triv = lax.shift_right_logical(tvv, _i32(1))
                b_i2[sl] = jnp.where(res & (ordv == 0), slotv, -1)
                b_i3[sl] = jnp.where(res & (ordv == 1), slotv, -1)
                b_val[sl] = tag + triv + 1
                keep = mask & jnp.logical_not(res)
                ki = jnp.where(keep, _i32(1), _i32(0))
                kci = plsc.cumsum(ki)
                b_i4[sl] = jnp.where(keep, wid * _CAP + nn + kci - 1, -1)
                nn = nn + jnp.sum(ki, dtype=_i32)
                wi = jnp.where(iswin, _i32(1), _i32(0))
                wci = plsc.cumsum(wi)
                b_i5[sl] = jnp.where(iswin, wid * _CAP + wc0 + wci - 1, -1)
                wc0 = wc0 + jnp.sum(wi, dtype=_i32)
                return (nn, wc0)
            newn, wn = lax.fori_loop(_i32(0), _i32(NV), v2, (newn, wn))
            d1 = pltpu.async_copy(
                b_val, T0.at[plsc.Indices(b_i2, ignored_value=-1)], s1)
            d2 = pltpu.async_copy(
                b_val, T1.at[plsc.Indices(b_i3, ignored_value=-1)], s2)
            d3 = pltpu.async_copy(
                b_g, ag.at[plsc.Indices(b_i4, ignored_value=-1)], s3)
            d4 = pltpu.async_copy(
                b_c0, ac0.at[plsc.Indices(b_i4, ignored_value=-1)], s4)
            d5 = pltpu.async_copy(
                b_c1, ac1.at[plsc.Indices(b_i4, ignored_value=-1)], s5)
            d1.wait()
            d2.wait()
            d1 = pltpu.async_copy(
                b_tv, atv.at[plsc.Indices(b_i4, ignored_value=-1)], s1)
            d2 = pltpu.async_copy(
                b_slot, wslots.at[plsc.Indices(b_i5, ignored_value=-1)], s2)
            d1.wait()
            d2.wait()
            d3.wait()
            d4.wait()
            d5.wait()
            return (newn, wn)
        return lax.fori_loop(_i32(0), nwin, win, (_i32(0), _i32(0)))

    def phase_d(wn, tag, acc):
        nwin = (wn + _W - 1) // _W

        def win(w, acc):
            off = w * _W
            pltpu.sync_copy(wslots.at[pl.ds(wid * _CAP + off, _W)], b_slot)
            rem = wn - off

            def v1(j, _):
                sl = pl.ds(j * 16, 16)
                lane = j * 16 + LANE
                b_i1[sl] = jnp.where(lane < rem, b_slot[sl], -1)
                return 0
            lax.fori_loop(_i32(0), _i32(NV), v1, 0)
            d1 = pltpu.async_copy(
                T0.at[plsc.Indices(b_i1, ignored_value=-1)], b_w, s1)
            d2 = pltpu.async_copy(
                T1.at[plsc.Indices(b_i1, ignored_value=-1)], b_w2, s2)
            d1.wait()
            d2.wait()

            def v2(j, _):
                sl = pl.ds(j * 16, 16)
                a0 = b_w[sl] - (tag + 1)
                a1 = b_w2[sl] - (tag + 1)
                lane = j * 16 + LANE
                mask = lane < rem
                t0 = jnp.where((a0 >= 0) & (a0 < _F), a0, 0)
                t1 = jnp.where((a1 >= 0) & (a1 < _F), a1, 0)
                b_i2[sl] = jnp.where(mask, t0, -1)
                b_i3[sl] = jnp.where(mask, t1, -1)
                return 0
            lax.fori_loop(_i32(0), _i32(NV), v2, 0)
            d1 = pltpu.async_copy(
                fnx.at[plsc.Indices(b_i2, ignored_value=-1)], b_x0, s0)
            d2 = pltpu.async_copy(
                fny.at[plsc.Indices(b_i2, ignored_value=-1)], b_y0, s1)
            d3 = pltpu.async_copy(
                fnz.at[plsc.Indices(b_i2, ignored_value=-1)], b_z0, s2)
            d4 = pltpu.async_copy(
                fnx.at[plsc.Indices(b_i3, ignored_value=-1)], b_x1, s3)
            d5 = pltpu.async_copy(
                fny.at[plsc.Indices(b_i3, ignored_value=-1)], b_y1, s4)
            d6 = pltpu.async_copy(
                fnz.at[plsc.Indices(b_i3, ignored_value=-1)], b_z1, s5)
            d1.wait()
            d2.wait()
            d3.wait()
            d4.wait()
            d5.wait()
            d6.wait()

            def v3(j, acc):
                sl = pl.ds(j * 16, 16)
                dot = (b_x0[sl] * b_x1[sl] + b_y0[sl] * b_y1[sl]
                       + b_z0[sl] * b_z1[sl])
                term = jnp.abs(1.0 - jnp.clip(dot, -1.0, 1.0))
                lane = j * 16 + LANE
                term = jnp.where(lane < rem, term, 0.0)
                return acc + term
            return lax.fori_loop(_i32(0), _i32(NV), v3, acc)
        return lax.fori_loop(_i32(0), nwin, win, acc)

    tag1 = _i32(_TAGSH)
    r1 = _i32(1)
    n1 = _i32(_PER_W)
    phase_a(c0p, c1p, None, wid * _CAP, n1, r1, tag1)
    plsc.subcore_barrier()
    n2, wn = phase_bc(c0p, c1p, tvp, None, wid * _CAP, n1, r1, tag1)
    plsc.subcore_barrier()
    acc = phase_d(wn, tag1, jnp.zeros((16,), jnp.float32))
    ucnt = wn
    plsc.subcore_barrier()

    def round_body(r, carry):
        n, ucnt, acc = carry
        tag = r * _TAGSH
        phase_a(ac0, ac1, ag, wid * _CAP, n, r, tag)
        plsc.subcore_barrier()
        n2, wn = phase_bc(ac0, ac1, atv, ag, wid * _CAP, n, r, tag)
        plsc.subcore_barrier()
        acc = phase_d(wn, tag, acc)
        plsc.subcore_barrier()
        return (n2, ucnt + wn, acc)
    n, ucnt, acc = lax.fori_loop(
        _i32(2), _i32(_R + 1), round_body, (n2, ucnt, acc))

    psum = jnp.sum(acc)
    pcnt = ucnt.astype(jnp.float32)
    b_red[...] = jnp.where(LANE == 0, psum,
                           jnp.where(LANE == 1, pcnt, 0.0))
    pltpu.sync_copy(b_red, shared.at[wid])
    plsc.subcore_barrier()

    @pl.when(wid == 0)
    def _():
        def rb(i, a):
            pltpu.sync_copy(shared.at[i], b_red2)
            return a + b_red2[...]
        tot = lax.fori_loop(_i32(0), _i32(_NW), rb, jnp.zeros((16,), jnp.float32))
        b_red2[...] = tot
        pltpu.sync_copy(b_red2, out)


def kernel(face_normals, t_pos_idx):
    a = t_pos_idx.astype(_i32)
    v0, v1, v2 = a[:, 0], a[:, 1], a[:, 2]
    ea = jnp.stack([v0, v1, v2], axis=0).reshape(-1)
    eb = jnp.stack([v1, v2, v0], axis=0).reshape(-1)
    c0f = jnp.minimum(ea, eb)
    c1f = jnp.maximum(ea, eb)
    order = (ea > eb).astype(_i32)
    tri = jnp.tile(jnp.arange(_F, dtype=_i32), 3)
    tvf = 2 * tri + order

    def padw(x):
        return jnp.pad(x.reshape(_NW, _PER_W),
                       ((0, 0), (0, _CAP - _PER_W))).reshape(-1)

    fn = face_normals.astype(jnp.float32)
    mesh = plsc.VectorSubcoreMesh(
        core_axis_name="c", subcore_axis_name="s", num_cores=1)
    out = pl.kernel(
        _body,
        out_type=jax.ShapeDtypeStruct((16,), jnp.float32),
        mesh=mesh,
        compiler_params=pltpu.CompilerParams(needs_layout_passes=False),
        scratch_types=[
            pltpu.HBM((_M,), _i32),
            pltpu.HBM((_M,), _i32),
            pltpu.HBM((_M,), _i32),
            pltpu.HBM((_NW * _CAP,), _i32),
            pltpu.HBM((_NW * _CAP,), _i32),
            pltpu.HBM((_NW * _CAP,), _i32),
            pltpu.HBM((_NW * _CAP,), _i32),
            pltpu.HBM((_NW * _CAP,), _i32),
            pltpu.HBM((_NW * _CAP,), _i32),
            pltpu.VMEM((_W,), _i32),
            pltpu.VMEM((_W,), _i32),
            pltpu.VMEM((_W,), _i32),
            pltpu.VMEM((_W,), _i32),
            pltpu.VMEM((_W,), _i32),
            pltpu.VMEM((_W,), _i32),
            pltpu.VMEM((_W,), _i32),
            pltpu.VMEM((_W,), _i32),
            pltpu.VMEM((_W,), _i32),
            pltpu.VMEM((_W,), _i32),
            pltpu.VMEM((_W,), _i32),
            pltpu.VMEM((_W,), _i32),
            pltpu.VMEM((_W,), _i32),
            pltpu.VMEM((_W,), _i32),
            pltpu.VMEM((_W,), jnp.float32),
            pltpu.VMEM((_W,), jnp.float32),
            pltpu.VMEM((_W,), jnp.float32),
            pltpu.VMEM((_W,), jnp.float32),
            pltpu.VMEM((_W,), jnp.float32),
            pltpu.VMEM((_W,), jnp.float32),
            pltpu.VMEM((16,), jnp.float32),
            pltpu.VMEM((16,), jnp.float32),
            pltpu.VMEM((8192,), _i32),
            pltpu.VMEM_SHARED((_NW, 16), jnp.float32),
            pltpu.SemaphoreType.DMA,
            pltpu.SemaphoreType.DMA,
            pltpu.SemaphoreType.DMA,
            pltpu.SemaphoreType.DMA,
            pltpu.SemaphoreType.DMA,
            pltpu.SemaphoreType.DMA,
        ],
    )(c0f, c1f, padw(c0f), padw(c1f), padw(tvf),
      fn[:, 0], fn[:, 1], fn[:, 2])
    return out[0] / out[1]
